# speedup vs baseline: 2.9597x; 2.9597x over previous
import functools

import jax
import jax.numpy as jnp
from jax import lax
from jax.experimental import pallas as pl
from jax.experimental.pallas import tpu as pltpu
from jax.experimental.pallas import tpu_sc as plsc

D = 128
B = 4096 * 50
NW = 32
CHUNK = 128
PER_W = B // NW
NCH = PER_W // CHUNK

_mesh = plsc.VectorSubcoreMesh(core_axis_name="c", subcore_axis_name="s")


@functools.partial(
    pl.kernel,
    mesh=_mesh,
    out_type=jax.ShapeDtypeStruct((B, D), jnp.float32),
    scratch_types=[
        pltpu.VMEM((NCH, CHUNK), jnp.int32),
        pltpu.VMEM((CHUNK, D), jnp.float32),
        pltpu.SemaphoreType.DMA,
    ],
)
def _gather_kernel(idx_hbm, table_hbm, out_hbm, idx_v, rows_v, sem):
    wid = lax.axis_index("s") * 2 + lax.axis_index("c")
    base = wid * PER_W
    pltpu.sync_copy(idx_hbm.at[wid], idx_v)

    def body(j, carry):
        pltpu.async_copy(table_hbm.at[idx_v.at[j]], rows_v, sem).wait()
        pltpu.sync_copy(rows_v, out_hbm.at[pl.ds(base + j * CHUNK, CHUNK)])
        return carry

    lax.fori_loop(0, NCH, body, 0)


def kernel(x, table):
    idx = x.astype(jnp.int32).reshape(NW, NCH, CHUNK)
    out = _gather_kernel(idx, table)
    return out.reshape(x.shape[0], x.shape[1], D)


# speedup vs baseline: 3.3171x; 1.1207x over previous
import functools

import jax
import jax.numpy as jnp
from jax import lax
from jax.experimental import pallas as pl
from jax.experimental.pallas import tpu as pltpu
from jax.experimental.pallas import tpu_sc as plsc

D = 128
B = 4096 * 50
NW = 32
CHUNK = 128
PER_W = B // NW
NCH = PER_W // CHUNK

_mesh = plsc.VectorSubcoreMesh(core_axis_name="c", subcore_axis_name="s")


@functools.partial(
    pl.kernel,
    mesh=_mesh,
    out_type=jax.ShapeDtypeStruct((B, D), jnp.float32),
    scratch_types=[
        pltpu.VMEM((NCH, CHUNK), jnp.int32),
        pltpu.VMEM((CHUNK, D), jnp.float32),
        pltpu.VMEM((CHUNK, D), jnp.float32),
        pltpu.SemaphoreType.DMA,
        pltpu.SemaphoreType.DMA,
        pltpu.SemaphoreType.DMA,
        pltpu.SemaphoreType.DMA,
    ],
)
def _gather_kernel(idx_hbm, table_hbm, out_hbm, idx_v, rows0, rows1,
                   gs0, gs1, ss0, ss1):
    wid = lax.axis_index("s") * 2 + lax.axis_index("c")
    base = wid * PER_W
    pltpu.sync_copy(idx_hbm.at[wid], idx_v)

    def gather(j, buf, sem):
        return pltpu.make_async_copy(table_hbm.at[idx_v.at[j]], buf, sem)

    def scatter(j, buf, sem):
        return pltpu.make_async_copy(
            buf, out_hbm.at[pl.ds(base + j * CHUNK, CHUNK)], sem)

    gather(0, rows0, gs0).start()
    gather(1, rows1, gs1).start()
    gather(0, rows0, gs0).wait()
    scatter(0, rows0, ss0).start()

    def body(g, carry):
        j = 2 * g + 1
        scatter(j - 1, rows0, ss0).wait()
        gather(j + 1, rows0, gs0).start()
        gather(j, rows1, gs1).wait()
        scatter(j, rows1, ss1).start()

        scatter(j, rows1, ss1).wait()
        gather(j + 2, rows1, gs1).start()
        gather(j + 1, rows0, gs0).wait()
        scatter(j + 1, rows0, ss0).start()
        return carry

    lax.fori_loop(0, NCH // 2 - 1, body, 0)

    j_last = NCH - 1
    gather(j_last, rows1, gs1).wait()
    scatter(j_last, rows1, ss1).start()
    scatter(j_last - 1, rows0, ss0).wait()
    scatter(j_last, rows1, ss1).wait()


def kernel(x, table):
    idx = x.astype(jnp.int32).reshape(NW, NCH, CHUNK)
    out = _gather_kernel(idx, table)
    return out.reshape(x.shape[0], x.shape[1], D)


# speedup vs baseline: 5.1346x; 1.5479x over previous
import functools

import jax
import jax.numpy as jnp
from jax import lax
from jax.experimental import pallas as pl
from jax.experimental.pallas import tpu as pltpu
from jax.experimental.pallas import tpu_sc as plsc

BATCH = 4096
HIST = 50
D = 128
NW = 32
PER_W = BATCH // NW

_mesh = plsc.VectorSubcoreMesh(core_axis_name="c", subcore_axis_name="s")


@functools.partial(
    pl.kernel,
    mesh=_mesh,
    out_type=jax.ShapeDtypeStruct((BATCH, HIST, D), jnp.float32),
    scratch_types=[
        pltpu.VMEM((PER_W, HIST), jnp.int32),
        pltpu.VMEM((HIST, D), jnp.float32),
        pltpu.VMEM((HIST, D), jnp.float32),
        pltpu.SemaphoreType.DMA,
        pltpu.SemaphoreType.DMA,
        pltpu.SemaphoreType.DMA,
        pltpu.SemaphoreType.DMA,
    ],
)
def _gather_kernel(idx_hbm, table_hbm, out_hbm, idx_v, rows0, rows1,
                   gs0, gs1, ss0, ss1):
    wid = lax.axis_index("s") * 2 + lax.axis_index("c")
    base = wid * PER_W
    pltpu.sync_copy(idx_hbm.at[pl.ds(base, PER_W)], idx_v)

    def gather(j, buf, sem):
        return pltpu.make_async_copy(table_hbm.at[idx_v.at[j]], buf, sem)

    def scatter(j, buf, sem):
        return pltpu.make_async_copy(buf, out_hbm.at[base + j], sem)

    gather(0, rows0, gs0).start()
    gather(1, rows1, gs1).start()
    gather(0, rows0, gs0).wait()
    scatter(0, rows0, ss0).start()

    def body(g, carry):
        j = 2 * g + 1
        scatter(j - 1, rows0, ss0).wait()
        gather(j + 1, rows0, gs0).start()
        gather(j, rows1, gs1).wait()
        scatter(j, rows1, ss1).start()

        scatter(j, rows1, ss1).wait()
        gather(j + 2, rows1, gs1).start()
        gather(j + 1, rows0, gs0).wait()
        scatter(j + 1, rows0, ss0).start()
        return carry

    lax.fori_loop(0, PER_W // 2 - 1, body, 0)

    j_last = PER_W - 1
    gather(j_last, rows1, gs1).wait()
    scatter(j_last, rows1, ss1).start()
    scatter(j_last - 1, rows0, ss0).wait()
    scatter(j_last, rows1, ss1).wait()


def kernel(x, table):
    return _gather_kernel(x.astype(jnp.int32), table)


# speedup vs baseline: 5.9359x; 1.1561x over previous
import functools

import jax
import jax.numpy as jnp
from jax import lax
from jax.experimental import pallas as pl
from jax.experimental.pallas import tpu as pltpu
from jax.experimental.pallas import tpu_sc as plsc

BATCH = 4096
HIST = 50
D = 128
NW = 32
PER_W = BATCH // NW
CB = 8
NST = PER_W // CB

_mesh = plsc.VectorSubcoreMesh(core_axis_name="c", subcore_axis_name="s")


@functools.partial(
    pl.kernel,
    mesh=_mesh,
    out_type=jax.ShapeDtypeStruct((BATCH, HIST, D), jnp.float32),
    scratch_types=[
        pltpu.VMEM((PER_W, HIST), jnp.int32),
        pltpu.VMEM((CB, HIST, D), jnp.float32),
        pltpu.VMEM((CB, HIST, D), jnp.float32),
        pltpu.SemaphoreType.DMA,
        pltpu.SemaphoreType.DMA,
        pltpu.SemaphoreType.DMA,
        pltpu.SemaphoreType.DMA,
    ],
)
def _gather_kernel(idx_hbm, table_hbm, out_hbm, idx_v, rows0, rows1,
                   gs0, gs1, ss0, ss1):
    wid = lax.axis_index("s") * 2 + lax.axis_index("c")
    base = wid * PER_W
    pltpu.sync_copy(idx_hbm.at[pl.ds(base, PER_W)], idx_v)

    class gather:
        def __init__(self, j, buf, sem):
            self.copies = [
                pltpu.make_async_copy(
                    table_hbm.at[idx_v.at[j * CB + b]], buf.at[b], sem)
                for b in range(CB)
            ]

        def start(self):
            for c in self.copies:
                c.start()

        def wait(self):
            for c in self.copies:
                c.wait()

    def scatter(j, buf, sem):
        return pltpu.make_async_copy(
            buf, out_hbm.at[pl.ds(base + j * CB, CB)], sem)

    gather(0, rows0, gs0).start()
    gather(1, rows1, gs1).start()
    gather(0, rows0, gs0).wait()
    scatter(0, rows0, ss0).start()

    def body(g, carry):
        j = 2 * g + 1
        scatter(j - 1, rows0, ss0).wait()
        gather(j + 1, rows0, gs0).start()
        gather(j, rows1, gs1).wait()
        scatter(j, rows1, ss1).start()

        scatter(j, rows1, ss1).wait()
        gather(j + 2, rows1, gs1).start()
        gather(j + 1, rows0, gs0).wait()
        scatter(j + 1, rows0, ss0).start()
        return carry

    lax.fori_loop(0, NST // 2 - 1, body, 0)

    j_last = NST - 1
    gather(j_last, rows1, gs1).wait()
    scatter(j_last, rows1, ss1).start()
    scatter(j_last - 1, rows0, ss0).wait()
    scatter(j_last, rows1, ss1).wait()


def kernel(x, table):
    return _gather_kernel(x.astype(jnp.int32), table)


# speedup vs baseline: 5.9466x; 1.0018x over previous
import functools

import jax
import jax.numpy as jnp
from jax import lax
from jax.experimental import pallas as pl
from jax.experimental.pallas import tpu as pltpu
from jax.experimental.pallas import tpu_sc as plsc

BATCH = 4096
HIST = 50
D = 128
NW = 32
PER_W = BATCH // NW
CB = 8
NST = PER_W // CB

_mesh = plsc.VectorSubcoreMesh(core_axis_name="c", subcore_axis_name="s")


@functools.partial(
    pl.kernel,
    mesh=_mesh,
    out_type=jax.ShapeDtypeStruct((BATCH, HIST, D), jnp.float32),
    compiler_params=pltpu.CompilerParams(use_tc_tiling_on_sc=True),
    scratch_types=[
        pltpu.VMEM((PER_W, HIST), jnp.int32),
        pltpu.VMEM((CB, HIST, D), jnp.float32),
        pltpu.VMEM((CB, HIST, D), jnp.float32),
        pltpu.SemaphoreType.DMA,
        pltpu.SemaphoreType.DMA,
        pltpu.SemaphoreType.DMA,
        pltpu.SemaphoreType.DMA,
    ],
)
def _gather_kernel(idx_hbm, table_hbm, out_hbm, idx_v, rows0, rows1,
                   gs0, gs1, ss0, ss1):
    wid = lax.axis_index("s") * 2 + lax.axis_index("c")
    base = wid * PER_W
    pltpu.sync_copy(idx_hbm.at[pl.ds(base, PER_W)], idx_v)

    class gather:
        def __init__(self, j, buf, sem):
            self.copies = [
                pltpu.make_async_copy(
                    table_hbm.at[idx_v.at[j * CB + b]], buf.at[b], sem)
                for b in range(CB)
            ]

        def start(self):
            for c in self.copies:
                c.start()

        def wait(self):
            for c in self.copies:
                c.wait()

    def scatter(j, buf, sem):
        return pltpu.make_async_copy(
            buf, out_hbm.at[pl.ds(base + j * CB, CB)], sem)

    gather(0, rows0, gs0).start()
    gather(1, rows1, gs1).start()
    gather(0, rows0, gs0).wait()
    scatter(0, rows0, ss0).start()

    def body(g, carry):
        j = 2 * g + 1
        scatter(j - 1, rows0, ss0).wait()
        gather(j + 1, rows0, gs0).start()
        gather(j, rows1, gs1).wait()
        scatter(j, rows1, ss1).start()

        scatter(j, rows1, ss1).wait()
        gather(j + 2, rows1, gs1).start()
        gather(j + 1, rows0, gs0).wait()
        scatter(j + 1, rows0, ss0).start()
        return carry

    lax.fori_loop(0, NST // 2 - 1, body, 0)

    j_last = NST - 1
    gather(j_last, rows1, gs1).wait()
    scatter(j_last, rows1, ss1).start()
    scatter(j_last - 1, rows0, ss0).wait()
    scatter(j_last, rows1, ss1).wait()


def kernel(x, table):
    return _gather_kernel(x.astype(jnp.int32), table)


# speedup vs baseline: 10.4422x; 1.7560x over previous
import functools

import jax
import jax.numpy as jnp
from jax import lax
from jax.experimental import pallas as pl
from jax.experimental.pallas import tpu as pltpu
from jax.experimental.pallas import tpu_sc as plsc

BATCH = 4096
HIST = 50
D = 128
NW = 32
PER_W = BATCH // NW
CB = 8
NST = PER_W // CB

_mesh = plsc.VectorSubcoreMesh(core_axis_name="c", subcore_axis_name="s")


@functools.partial(
    pl.kernel,
    mesh=_mesh,
    out_type=jax.ShapeDtypeStruct((HIST, BATCH, D), jnp.float32),
    scratch_types=[
        pltpu.VMEM((PER_W, HIST), jnp.int32),
        pltpu.VMEM((CB, HIST, D), jnp.float32),
        pltpu.VMEM((CB, HIST, D), jnp.float32),
        pltpu.SemaphoreType.DMA,
        pltpu.SemaphoreType.DMA,
        pltpu.SemaphoreType.DMA,
        pltpu.SemaphoreType.DMA,
    ],
)
def _gather_kernel(idx_hbm, table_hbm, out_hbm, idx_v, rows0, rows1,
                   gs0, gs1, ss0, ss1):
    wid = lax.axis_index("s") * 2 + lax.axis_index("c")
    base = wid * PER_W
    pltpu.sync_copy(idx_hbm.at[pl.ds(base, PER_W)], idx_v)

    class gather:
        def __init__(self, j, buf, sem):
            self.copies = [
                pltpu.make_async_copy(
                    table_hbm.at[idx_v.at[j * CB + b]], buf.at[b], sem)
                for b in range(CB)
            ]

        def start(self):
            for c in self.copies:
                c.start()

        def wait(self):
            for c in self.copies:
                c.wait()

    class scatter:
        def __init__(self, j, buf, sem):
            self.copies = [
                pltpu.make_async_copy(
                    buf.at[b], out_hbm.at[:, base + j * CB + b, :], sem)
                for b in range(CB)
            ]

        def start(self):
            for c in self.copies:
                c.start()

        def wait(self):
            for c in self.copies:
                c.wait()

    gather(0, rows0, gs0).start()
    gather(1, rows1, gs1).start()
    gather(0, rows0, gs0).wait()
    scatter(0, rows0, ss0).start()

    def body(g, carry):
        j = 2 * g + 1
        scatter(j - 1, rows0, ss0).wait()
        gather(j + 1, rows0, gs0).start()
        gather(j, rows1, gs1).wait()
        scatter(j, rows1, ss1).start()

        scatter(j, rows1, ss1).wait()
        gather(j + 2, rows1, gs1).start()
        gather(j + 1, rows0, gs0).wait()
        scatter(j + 1, rows0, ss0).start()
        return carry

    lax.fori_loop(0, NST // 2 - 1, body, 0)

    j_last = NST - 1
    gather(j_last, rows1, gs1).wait()
    scatter(j_last, rows1, ss1).start()
    scatter(j_last - 1, rows0, ss0).wait()
    scatter(j_last, rows1, ss1).wait()


def kernel(x, table):
    out_t = _gather_kernel(x.astype(jnp.int32), table)
    return out_t.transpose(1, 0, 2)
